# Initial kernel scaffold; baseline (speedup 1.0000x reference)
import functools

import jax
import jax.numpy as jnp
from jax import lax
from jax.experimental import pallas as pl
from jax.experimental.pallas import tpu as pltpu
from jax.experimental.pallas import tpu_sc as plsc

_N = 10000
_E = 320000
_H = 128
_R = 5
_NQ = 4
_QW = 32
_NS = 16
_NC = 2
_CH = 128
_CPT = 157
_EPT = _CH * _CPT
_EP = _NS * _EPT
_RN = _R * _N
_STRIPE = _RN // _NS
_ZR = 125
_BM = 400


def _layer_norm(x, g, b):
    mu = jnp.mean(x, axis=-1, keepdims=True)
    var = jnp.mean(jnp.square(x - mu), axis=-1, keepdims=True)
    return (x - mu) / jnp.sqrt(var + 1e-5) * g + b


def _gelu(x):
    return jax.nn.gelu(x, approximate=False)


def _proj_body(x_ref, w_ref, b_ref, g_ref, be_ref, o_ref):
    h = jnp.dot(x_ref[...], w_ref[...], preferred_element_type=jnp.float32)
    o_ref[...] = _gelu(_layer_norm(h + b_ref[...], g_ref[...], be_ref[...]))


def _proj(x, w, b, g, be):
    return pl.pallas_call(
        _proj_body,
        grid=(_N // _BM,),
        in_specs=[
            pl.BlockSpec((_BM, _H), lambda i: (i, 0)),
            pl.BlockSpec((_H, _H), lambda i: (0, 0)),
            pl.BlockSpec((1, _H), lambda i: (0, 0)),
            pl.BlockSpec((1, _H), lambda i: (0, 0)),
            pl.BlockSpec((1, _H), lambda i: (0, 0)),
        ],
        out_specs=pl.BlockSpec((_BM, _H), lambda i: (i, 0)),
        out_shape=jax.ShapeDtypeStruct((_N, _H), jnp.float32),
    )(x, w, b.reshape(1, -1), g.reshape(1, -1), be.reshape(1, -1))


def _mid_body(a_ref, wq_ref, b_ref, o_ref):
    acc = jnp.zeros((_BM, _H), jnp.float32)
    for q in range(_NQ):
        acc += jnp.dot(a_ref[q], wq_ref[q], preferred_element_type=jnp.float32)
    o_ref[...] = _gelu(acc + b_ref[...])


def _mid(a2, wq, b):
    rw = _R * _QW
    return pl.pallas_call(
        _mid_body,
        grid=(_N // _BM,),
        in_specs=[
            pl.BlockSpec((_NQ, _BM, rw), lambda i: (0, i, 0)),
            pl.BlockSpec((_NQ, rw, _H), lambda i: (0, 0, 0)),
            pl.BlockSpec((1, _H), lambda i: (0, 0)),
        ],
        out_specs=pl.BlockSpec((_BM, _H), lambda i: (i, 0)),
        out_shape=jax.ShapeDtypeStruct((_N, _H), jnp.float32),
    )(a2, wq, b.reshape(1, -1))


def _head_body(h_ref, w1_ref, b1_ref, g1_ref, bb1_ref, w2_ref, b2_ref,
               wt1_ref, bt1_ref, gt1_ref, bbt1_ref, wt2_ref, bt2_ref,
               cat_ref, tax_ref):
    h = h_ref[...]
    z = _gelu(_layer_norm(
        jnp.dot(h, w1_ref[...], preferred_element_type=jnp.float32) + b1_ref[...],
        g1_ref[...], bb1_ref[...]))
    cat_ref[...] = jnp.dot(z, w2_ref[...], preferred_element_type=jnp.float32) + b2_ref[...]
    t = _gelu(_layer_norm(
        jnp.dot(h, wt1_ref[...], preferred_element_type=jnp.float32) + bt1_ref[...],
        gt1_ref[...], bbt1_ref[...]))
    tax_ref[...] = jnp.dot(t, wt2_ref[...], preferred_element_type=jnp.float32) + bt2_ref[...]


def _head(h, w1, b1, g1, bb1, w2, b2, wt1, bt1, gt1, bbt1, wt2, bt2):
    h2 = 2 * _H
    out = 400
    tax = 20
    return pl.pallas_call(
        _head_body,
        grid=(_N // _BM,),
        in_specs=[
            pl.BlockSpec((_BM, _H), lambda i: (i, 0)),
            pl.BlockSpec((_H, h2), lambda i: (0, 0)),
            pl.BlockSpec((1, h2), lambda i: (0, 0)),
            pl.BlockSpec((1, h2), lambda i: (0, 0)),
            pl.BlockSpec((1, h2), lambda i: (0, 0)),
            pl.BlockSpec((h2, out), lambda i: (0, 0)),
            pl.BlockSpec((1, out), lambda i: (0, 0)),
            pl.BlockSpec((_H, h2), lambda i: (0, 0)),
            pl.BlockSpec((1, h2), lambda i: (0, 0)),
            pl.BlockSpec((1, h2), lambda i: (0, 0)),
            pl.BlockSpec((1, h2), lambda i: (0, 0)),
            pl.BlockSpec((h2, tax), lambda i: (0, 0)),
            pl.BlockSpec((1, tax), lambda i: (0, 0)),
        ],
        out_specs=[
            pl.BlockSpec((_BM, out), lambda i: (i, 0)),
            pl.BlockSpec((_BM, tax), lambda i: (i, 0)),
        ],
        out_shape=[
            jax.ShapeDtypeStruct((_N, out), jnp.float32),
            jax.ShapeDtypeStruct((_N, tax), jnp.float32),
        ],
    )(h, w1, b1.reshape(1, -1), g1.reshape(1, -1), bb1.reshape(1, -1),
      w2, b2.reshape(1, -1), wt1, bt1.reshape(1, -1), gt1.reshape(1, -1),
      bbt1.reshape(1, -1), wt2, bt2.reshape(1, -1))


def _splat16(v, i):
    return lax.gather(
        v,
        jnp.full((16, 1), i, jnp.int32),
        lax.GatherDimensionNumbers(
            offset_dims=(), collapsed_slice_dims=(0,), start_index_map=(0,)),
        (1,),
        mode=lax.GatherScatterMode.PROMISE_IN_BOUNDS)


def _sc_body(hqf, src_all, cidx_t, attr_t, aq,
             src_v, cidx_v, attr_v, rows_v, zbuf, acc, sem):
    c = lax.axis_index("c")
    s = lax.axis_index("s")

    z16 = jnp.zeros((16,), jnp.float32)
    for i in range(_ZR):
        zbuf[i, 0:16] = z16
        zbuf[i, 16:32] = z16

    pltpu.sync_copy(cidx_t.at[s], cidx_v)
    pltpu.sync_copy(attr_t.at[s], attr_v)

    for qi in range(2):
        q = 2 * c + qi
        pltpu.sync_copy(src_all.at[q, s], src_v)

        def _zero(k, carry):
            pltpu.sync_copy(zbuf, acc.at[pl.ds(s * _STRIPE + k * _ZR, _ZR)])
            return carry
        lax.fori_loop(0, _STRIPE // _ZR, _zero, 0)
        plsc.subcore_barrier()

        def _chunk(j, carry):
            pltpu.async_copy(hqf.at[src_v.at[j]], rows_v, sem).wait()
            for g in range(_CH // 16):
                a16 = attr_v[j, pl.ds(g * 16, 16)]
                for i in range(16):
                    e = g * 16 + i
                    spl = _splat16(a16, i)
                    rows_v[e, 0:16] = rows_v[e, 0:16] * spl
                    rows_v[e, 16:32] = rows_v[e, 16:32] * spl
            pltpu.sync_copy(rows_v, acc.at[cidx_v.at[j]], add=True)
            return carry
        lax.fori_loop(0, _CPT, _chunk, 0)
        plsc.subcore_barrier()

        pltpu.sync_copy(acc.at[pl.ds(s * _STRIPE, _STRIPE)],
                        aq.at[q, pl.ds(s * _STRIPE, _STRIPE)])
        plsc.subcore_barrier()


def _sc_agg(hqf, src_all, cidx_t, attr_t):
    kern = pl.kernel(
        _sc_body,
        out_type=jax.ShapeDtypeStruct((_NQ, _RN, _QW), jnp.float32),
        mesh=plsc.VectorSubcoreMesh(core_axis_name="c", subcore_axis_name="s"),
        scratch_types=[
            pltpu.VMEM((_CPT, _CH), jnp.int32),
            pltpu.VMEM((_CPT, _CH), jnp.int32),
            pltpu.VMEM((_CPT, _CH), jnp.float32),
            pltpu.VMEM((_CH, _QW), jnp.float32),
            pltpu.VMEM((_ZR, _QW), jnp.float32),
            pltpu.VMEM_SHARED((_RN, _QW), jnp.float32),
            pltpu.SemaphoreType.DMA,
        ],
    )
    return kern(hqf, src_all, cidx_t, attr_t)


def kernel(x, edge_index, edge_type, edge_attr, W_in, b_in, g_in, be_in,
           Wg, bg, W1, b1, g1, bb1, W2, b2, Wt1, bt1, gt1, bbt1, Wt2, bt2):
    src = edge_index[0]
    dst = edge_index[1]
    cidx = dst * _R + edge_type
    attr = edge_attr[:, 0]

    pad = _EP - _E
    srcp = jnp.pad(src, (0, pad))
    cidxp = jnp.pad(cidx, (0, pad))
    attrp = jnp.pad(attr, (0, pad))

    qoff = (jnp.arange(_NQ, dtype=jnp.int32) * _N)[:, None]
    src_all = (srcp[None, :] + qoff).reshape(_NQ, _NS, _CPT, _CH)
    cidx_t = cidxp.reshape(_NS, _CPT, _CH)
    attr_t = attrp.reshape(_NS, _CPT, _CH)

    h = _proj(x, W_in, b_in, g_in, be_in)

    for l in range(2):
        hqf = h.reshape(_N, _NQ, _QW).transpose(1, 0, 2).reshape(_NQ * _N, _QW)
        a = _sc_agg(hqf, src_all, cidx_t, attr_t)
        a2 = a.reshape(_NQ, _N, _R * _QW)
        wq = jnp.stack([
            Wg[l, :, qq * _QW:(qq + 1) * _QW, :].reshape(_R * _QW, _H)
            for qq in range(_NQ)])
        h = _mid(a2, wq, bg[l])

    return _head(h, W1, b1, g1, bb1, W2, b2, Wt1, bt1, gt1, bbt1, Wt2, bt2)


# speedup vs baseline: 5.9824x; 5.9824x over previous
import functools

import jax
import jax.numpy as jnp
import numpy as np
from jax import lax
from jax.experimental import pallas as pl
from jax.experimental.pallas import tpu as pltpu
from jax.experimental.pallas import tpu_sc as plsc

_N = 10000
_E = 320000
_H = 128
_R = 5
_NQ = 4
_QW = 32
_NS = 16
_NC = 2
_CH = 128
_CPT = 160
_CB = 16
_EPT = _CH * _CPT
_EP = _NS * _EPT
_RN = _R * _N
_STRIPE = 3128
_LSTRIPE = _RN - 15 * _STRIPE
_ZR = 184
_BM = 400


def _layer_norm(x, g, b):
    mu = jnp.mean(x, axis=-1, keepdims=True)
    var = jnp.mean(jnp.square(x - mu), axis=-1, keepdims=True)
    return (x - mu) / jnp.sqrt(var + 1e-5) * g + b


def _gelu(x):
    return x * 0.5 * (1.0 + lax.erf(x * np.float32(1.0 / np.sqrt(2.0))))


def _proj_body(x_ref, w_ref, b_ref, g_ref, be_ref, o_ref):
    h = jnp.dot(x_ref[...], w_ref[...], preferred_element_type=jnp.float32)
    o_ref[...] = _gelu(_layer_norm(h + b_ref[...], g_ref[...], be_ref[...]))


def _proj(x, w, b, g, be):
    return pl.pallas_call(
        _proj_body,
        grid=(_N // _BM,),
        in_specs=[
            pl.BlockSpec((_BM, _H), lambda i: (i, 0)),
            pl.BlockSpec((_H, _H), lambda i: (0, 0)),
            pl.BlockSpec((1, _H), lambda i: (0, 0)),
            pl.BlockSpec((1, _H), lambda i: (0, 0)),
            pl.BlockSpec((1, _H), lambda i: (0, 0)),
        ],
        out_specs=pl.BlockSpec((_BM, _H), lambda i: (i, 0)),
        out_shape=jax.ShapeDtypeStruct((_N, _H), jnp.float32),
    )(x, w, b.reshape(1, -1), g.reshape(1, -1), be.reshape(1, -1))


def _mid_body(a_ref, wq_ref, b_ref, o_ref):
    acc = jnp.zeros((_BM, _H), jnp.float32)
    for q in range(_NQ):
        acc += jnp.dot(a_ref[q], wq_ref[q], preferred_element_type=jnp.float32)
    o_ref[...] = _gelu(acc + b_ref[...])


def _mid(a2, wq, b):
    rw = _R * _QW
    return pl.pallas_call(
        _mid_body,
        grid=(_N // _BM,),
        in_specs=[
            pl.BlockSpec((_NQ, _BM, rw), lambda i: (0, i, 0)),
            pl.BlockSpec((_NQ, rw, _H), lambda i: (0, 0, 0)),
            pl.BlockSpec((1, _H), lambda i: (0, 0)),
        ],
        out_specs=pl.BlockSpec((_BM, _H), lambda i: (i, 0)),
        out_shape=jax.ShapeDtypeStruct((_N, _H), jnp.float32),
    )(a2, wq, b.reshape(1, -1))


def _head_body(h_ref, w1_ref, b1_ref, g1_ref, bb1_ref, w2_ref, b2_ref,
               wt1_ref, bt1_ref, gt1_ref, bbt1_ref, wt2_ref, bt2_ref,
               cat_ref, tax_ref):
    h = h_ref[...]
    z = _gelu(_layer_norm(
        jnp.dot(h, w1_ref[...], preferred_element_type=jnp.float32) + b1_ref[...],
        g1_ref[...], bb1_ref[...]))
    cat_ref[...] = jnp.dot(z, w2_ref[...], preferred_element_type=jnp.float32) + b2_ref[...]
    t = _gelu(_layer_norm(
        jnp.dot(h, wt1_ref[...], preferred_element_type=jnp.float32) + bt1_ref[...],
        gt1_ref[...], bbt1_ref[...]))
    tax_ref[...] = jnp.dot(t, wt2_ref[...], preferred_element_type=jnp.float32) + bt2_ref[...]


def _head(h, w1, b1, g1, bb1, w2, b2, wt1, bt1, gt1, bbt1, wt2, bt2):
    h2 = 2 * _H
    out = 400
    tax = 20
    return pl.pallas_call(
        _head_body,
        grid=(_N // _BM,),
        in_specs=[
            pl.BlockSpec((_BM, _H), lambda i: (i, 0)),
            pl.BlockSpec((_H, h2), lambda i: (0, 0)),
            pl.BlockSpec((1, h2), lambda i: (0, 0)),
            pl.BlockSpec((1, h2), lambda i: (0, 0)),
            pl.BlockSpec((1, h2), lambda i: (0, 0)),
            pl.BlockSpec((h2, out), lambda i: (0, 0)),
            pl.BlockSpec((1, out), lambda i: (0, 0)),
            pl.BlockSpec((_H, h2), lambda i: (0, 0)),
            pl.BlockSpec((1, h2), lambda i: (0, 0)),
            pl.BlockSpec((1, h2), lambda i: (0, 0)),
            pl.BlockSpec((1, h2), lambda i: (0, 0)),
            pl.BlockSpec((h2, tax), lambda i: (0, 0)),
            pl.BlockSpec((1, tax), lambda i: (0, 0)),
        ],
        out_specs=[
            pl.BlockSpec((_BM, out), lambda i: (i, 0)),
            pl.BlockSpec((_BM, tax), lambda i: (i, 0)),
        ],
        out_shape=[
            jax.ShapeDtypeStruct((_N, out), jnp.float32),
            jax.ShapeDtypeStruct((_N, tax), jnp.float32),
        ],
    )(h, w1, b1.reshape(1, -1), g1.reshape(1, -1), bb1.reshape(1, -1),
      w2, b2.reshape(1, -1), wt1, bt1.reshape(1, -1), gt1.reshape(1, -1),
      bbt1.reshape(1, -1), wt2, bt2.reshape(1, -1))


def _splat16(v, i):
    return lax.gather(
        v,
        jnp.full((16, 1), i, jnp.int32),
        lax.GatherDimensionNumbers(
            offset_dims=(), collapsed_slice_dims=(0,), start_index_map=(0,)),
        (1,),
        mode=lax.GatherScatterMode.PROMISE_IN_BOUNDS)


def _sc_body(hqf, src_all, cidx_t, attr_t, aq,
             src_v, cidx_v, attr_v, rows_v, zbuf, acc, sem):
    c = lax.axis_index("c")
    s = lax.axis_index("s")

    z16 = jnp.zeros((16,), jnp.float32)
    for i in range(_ZR):
        zbuf[i, 0:16] = z16
        zbuf[i, 16:32] = z16

    for qi in range(2):
        q = 2 * c + qi

        def _zero(k, carry):
            pltpu.sync_copy(zbuf, acc.at[pl.ds(s * _STRIPE + k * _ZR, _ZR)])
            return carry
        lax.fori_loop(0, jnp.where(s == _NS - 1, 16, 17), _zero, 0)

        @pl.when(s == _NS - 1)
        def _zero_tail():
            pltpu.sync_copy(zbuf.at[pl.ds(0, 136)],
                            acc.at[pl.ds(s * _STRIPE + 16 * _ZR, 136)])
        plsc.subcore_barrier()

        def _group(jg, carry):
            pltpu.sync_copy(src_all.at[q, s, pl.ds(jg * _CB, _CB)], src_v)
            pltpu.sync_copy(cidx_t.at[s, pl.ds(jg * _CB, _CB)], cidx_v)
            pltpu.sync_copy(attr_t.at[s, pl.ds(jg * _CB, _CB)], attr_v)

            def _chunk(j, carry2):
                pltpu.async_copy(hqf.at[src_v.at[j]], rows_v, sem).wait()
                for g in range(_CH // 16):
                    a16 = attr_v[j, pl.ds(g * 16, 16)]
                    for i in range(16):
                        e = g * 16 + i
                        spl = _splat16(a16, i)
                        rows_v[e, 0:16] = rows_v[e, 0:16] * spl
                        rows_v[e, 16:32] = rows_v[e, 16:32] * spl
                pltpu.sync_copy(rows_v, acc.at[cidx_v.at[j]], add=True)
                return carry2
            lax.fori_loop(0, _CB, _chunk, 0)
            return carry
        lax.fori_loop(0, _CPT // _CB, _group, 0)
        plsc.subcore_barrier()

        @pl.when(s < _NS - 1)
        def _writeout():
            pltpu.sync_copy(acc.at[pl.ds(s * _STRIPE, _STRIPE)],
                            aq.at[q, pl.ds(s * _STRIPE, _STRIPE)])

        @pl.when(s == _NS - 1)
        def _writeout_tail():
            pltpu.sync_copy(acc.at[pl.ds(s * _STRIPE, _LSTRIPE)],
                            aq.at[q, pl.ds(s * _STRIPE, _LSTRIPE)])
        plsc.subcore_barrier()


def _sc_agg(hqf, src_all, cidx_t, attr_t):
    kern = pl.kernel(
        _sc_body,
        out_type=jax.ShapeDtypeStruct((_NQ, _RN, _QW), jnp.float32),
        mesh=plsc.VectorSubcoreMesh(core_axis_name="c", subcore_axis_name="s"),
        compiler_params=pltpu.CompilerParams(use_tc_tiling_on_sc=False),
        scratch_types=[
            pltpu.VMEM((_CB, _CH), jnp.int32),
            pltpu.VMEM((_CB, _CH), jnp.int32),
            pltpu.VMEM((_CB, _CH), jnp.float32),
            pltpu.VMEM((_CH, _QW), jnp.float32),
            pltpu.VMEM((_ZR, _QW), jnp.float32),
            pltpu.VMEM_SHARED((_RN, _QW), jnp.float32),
            pltpu.SemaphoreType.DMA,
        ],
    )
    return kern(hqf, src_all, cidx_t, attr_t)


def kernel(x, edge_index, edge_type, edge_attr, W_in, b_in, g_in, be_in,
           Wg, bg, W1, b1, g1, bb1, W2, b2, Wt1, bt1, gt1, bbt1, Wt2, bt2):
    src = edge_index[0]
    dst = edge_index[1]
    cidx = dst * _R + edge_type
    attr = edge_attr[:, 0]

    pad = _EP - _E
    srcp = jnp.pad(src, (0, pad))
    cidxp = jnp.pad(cidx, (0, pad))
    attrp = jnp.pad(attr, (0, pad))

    qoff = (jnp.arange(_NQ, dtype=jnp.int32) * _N)[:, None]
    src_all = (srcp[None, :] + qoff).reshape(_NQ, _NS, _CPT, _CH)
    cidx_t = cidxp.reshape(_NS, _CPT, _CH)
    attr_t = attrp.reshape(_NS, _CPT, _CH)

    h = _proj(x, W_in, b_in, g_in, be_in)

    for l in range(2):
        hqf = h.reshape(_N, _NQ, _QW).transpose(1, 0, 2).reshape(_NQ * _N, _QW)
        a = _sc_agg(hqf, src_all, cidx_t, attr_t)
        a2 = a.reshape(_NQ, _N, _R * _QW)
        wq = jnp.stack([
            Wg[l, :, qq * _QW:(qq + 1) * _QW, :].reshape(_R * _QW, _H)
            for qq in range(_NQ)])
        h = _mid(a2, wq, bg[l])

    return _head(h, W1, b1, g1, bb1, W2, b2, Wt1, bt1, gt1, bbt1, Wt2, bt2)


# speedup vs baseline: 8.0314x; 1.3425x over previous
import functools

import jax
import jax.numpy as jnp
import numpy as np
from jax import lax
from jax.experimental import pallas as pl
from jax.experimental.pallas import tpu as pltpu
from jax.experimental.pallas import tpu_sc as plsc

_N = 10000
_E = 320000
_H = 128
_R = 5
_NQ = 4
_QW = 32
_NS = 16
_NC = 2
_CH = 128
_CPT = 160
_CB = 16
_EPT = _CH * _CPT
_EP = _NS * _EPT
_RN = _R * _N
_STRIPE = 3128
_LSTRIPE = _RN - 15 * _STRIPE
_ZR = 136
_NB = 4
_BM = 400


def _layer_norm(x, g, b):
    mu = jnp.mean(x, axis=-1, keepdims=True)
    var = jnp.mean(jnp.square(x - mu), axis=-1, keepdims=True)
    return (x - mu) / jnp.sqrt(var + 1e-5) * g + b


def _gelu(x):
    return x * 0.5 * (1.0 + lax.erf(x * np.float32(1.0 / np.sqrt(2.0))))


def _proj_body(x_ref, w_ref, b_ref, g_ref, be_ref, o_ref):
    h = jnp.dot(x_ref[...], w_ref[...], preferred_element_type=jnp.float32)
    o_ref[...] = _gelu(_layer_norm(h + b_ref[...], g_ref[...], be_ref[...]))


def _proj(x, w, b, g, be):
    return pl.pallas_call(
        _proj_body,
        grid=(_N // _BM,),
        in_specs=[
            pl.BlockSpec((_BM, _H), lambda i: (i, 0)),
            pl.BlockSpec((_H, _H), lambda i: (0, 0)),
            pl.BlockSpec((1, _H), lambda i: (0, 0)),
            pl.BlockSpec((1, _H), lambda i: (0, 0)),
            pl.BlockSpec((1, _H), lambda i: (0, 0)),
        ],
        out_specs=pl.BlockSpec((_BM, _H), lambda i: (i, 0)),
        out_shape=jax.ShapeDtypeStruct((_N, _H), jnp.float32),
    )(x, w, b.reshape(1, -1), g.reshape(1, -1), be.reshape(1, -1))


def _mid_body(a_ref, wq_ref, b_ref, o_ref):
    acc = jnp.zeros((_BM, _H), jnp.float32)
    for q in range(_NQ):
        acc += jnp.dot(a_ref[q], wq_ref[q], preferred_element_type=jnp.float32)
    o_ref[...] = _gelu(acc + b_ref[...])


def _mid(a2, wq, b):
    rw = _R * _QW
    return pl.pallas_call(
        _mid_body,
        grid=(_N // _BM,),
        in_specs=[
            pl.BlockSpec((_NQ, _BM, rw), lambda i: (0, i, 0)),
            pl.BlockSpec((_NQ, rw, _H), lambda i: (0, 0, 0)),
            pl.BlockSpec((1, _H), lambda i: (0, 0)),
        ],
        out_specs=pl.BlockSpec((_BM, _H), lambda i: (i, 0)),
        out_shape=jax.ShapeDtypeStruct((_N, _H), jnp.float32),
    )(a2, wq, b.reshape(1, -1))


def _head_body(h_ref, w1_ref, b1_ref, g1_ref, bb1_ref, w2_ref, b2_ref,
               wt1_ref, bt1_ref, gt1_ref, bbt1_ref, wt2_ref, bt2_ref,
               cat_ref, tax_ref):
    h = h_ref[...]
    z = _gelu(_layer_norm(
        jnp.dot(h, w1_ref[...], preferred_element_type=jnp.float32) + b1_ref[...],
        g1_ref[...], bb1_ref[...]))
    cat_ref[...] = jnp.dot(z, w2_ref[...], preferred_element_type=jnp.float32) + b2_ref[...]
    t = _gelu(_layer_norm(
        jnp.dot(h, wt1_ref[...], preferred_element_type=jnp.float32) + bt1_ref[...],
        gt1_ref[...], bbt1_ref[...]))
    tax_ref[...] = jnp.dot(t, wt2_ref[...], preferred_element_type=jnp.float32) + bt2_ref[...]


def _head(h, w1, b1, g1, bb1, w2, b2, wt1, bt1, gt1, bbt1, wt2, bt2):
    h2 = 2 * _H
    out = 400
    tax = 20
    return pl.pallas_call(
        _head_body,
        grid=(_N // _BM,),
        in_specs=[
            pl.BlockSpec((_BM, _H), lambda i: (i, 0)),
            pl.BlockSpec((_H, h2), lambda i: (0, 0)),
            pl.BlockSpec((1, h2), lambda i: (0, 0)),
            pl.BlockSpec((1, h2), lambda i: (0, 0)),
            pl.BlockSpec((1, h2), lambda i: (0, 0)),
            pl.BlockSpec((h2, out), lambda i: (0, 0)),
            pl.BlockSpec((1, out), lambda i: (0, 0)),
            pl.BlockSpec((_H, h2), lambda i: (0, 0)),
            pl.BlockSpec((1, h2), lambda i: (0, 0)),
            pl.BlockSpec((1, h2), lambda i: (0, 0)),
            pl.BlockSpec((1, h2), lambda i: (0, 0)),
            pl.BlockSpec((h2, tax), lambda i: (0, 0)),
            pl.BlockSpec((1, tax), lambda i: (0, 0)),
        ],
        out_specs=[
            pl.BlockSpec((_BM, out), lambda i: (i, 0)),
            pl.BlockSpec((_BM, tax), lambda i: (i, 0)),
        ],
        out_shape=[
            jax.ShapeDtypeStruct((_N, out), jnp.float32),
            jax.ShapeDtypeStruct((_N, tax), jnp.float32),
        ],
    )(h, w1, b1.reshape(1, -1), g1.reshape(1, -1), bb1.reshape(1, -1),
      w2, b2.reshape(1, -1), wt1, bt1.reshape(1, -1), gt1.reshape(1, -1),
      bbt1.reshape(1, -1), wt2, bt2.reshape(1, -1))


def _splat16(v, i):
    return lax.gather(
        v,
        jnp.full((16, 1), i, jnp.int32),
        lax.GatherDimensionNumbers(
            offset_dims=(), collapsed_slice_dims=(0,), start_index_map=(0,)),
        (1,),
        mode=lax.GatherScatterMode.PROMISE_IN_BOUNDS)


def _sc_body(hqf, src_all, cidx_t, attr_t, aq,
             src_v, cidx_v, attr_v, rows4, zbuf, acc, gsem, ssem):
    c = lax.axis_index("c")
    s = lax.axis_index("s")

    z16 = jnp.zeros((16,), jnp.float32)
    for i in range(_ZR):
        zbuf[i, 0:16] = z16
        zbuf[i, 16:32] = z16

    for qi in range(2):
        q = 2 * c + qi

        def _zero(k, carry):
            pltpu.sync_copy(zbuf, acc.at[pl.ds(s * _STRIPE + k * _ZR, _ZR)])
            return carry
        lax.fori_loop(0, jnp.where(s == _NS - 1, 22, 23), _zero, 0)

        @pl.when(s == _NS - 1)
        def _zero_tail():
            pltpu.sync_copy(zbuf.at[pl.ds(0, 88)],
                            acc.at[pl.ds(s * _STRIPE + 22 * _ZR, 88)])
        plsc.subcore_barrier()

        def _group(jg, carry):
            pltpu.sync_copy(src_all.at[q, s, pl.ds(jg * _CB, _CB)], src_v)
            pltpu.sync_copy(cidx_t.at[s, pl.ds(jg * _CB, _CB)], cidx_v)
            pltpu.sync_copy(attr_t.at[s, pl.ds(jg * _CB, _CB)], attr_v)

            def _quad(qq, carry2):
                gh = [pltpu.async_copy(
                          hqf.at[src_v.at[qq * _NB + b]], rows4.at[b], gsem)
                      for b in range(_NB)]
                sh = []
                for b in range(_NB):
                    jj = qq * _NB + b
                    gh[b].wait()
                    for g in range(_CH // 16):
                        a16 = attr_v[jj, pl.ds(g * 16, 16)]
                        for i in range(16):
                            e = g * 16 + i
                            spl = _splat16(a16, i)
                            rows4[b, e, 0:16] = rows4[b, e, 0:16] * spl
                            rows4[b, e, 16:32] = rows4[b, e, 16:32] * spl
                    sh.append(pltpu.async_copy(
                        rows4.at[b], acc.at[cidx_v.at[jj]], ssem, add=True))
                for h in sh:
                    h.wait()
                return carry2
            lax.fori_loop(0, _CB // _NB, _quad, 0)
            return carry
        lax.fori_loop(0, _CPT // _CB, _group, 0)
        plsc.subcore_barrier()

        @pl.when(s < _NS - 1)
        def _writeout():
            pltpu.sync_copy(acc.at[pl.ds(s * _STRIPE, _STRIPE)],
                            aq.at[q, pl.ds(s * _STRIPE, _STRIPE)])

        @pl.when(s == _NS - 1)
        def _writeout_tail():
            pltpu.sync_copy(acc.at[pl.ds(s * _STRIPE, _LSTRIPE)],
                            aq.at[q, pl.ds(s * _STRIPE, _LSTRIPE)])
        plsc.subcore_barrier()


def _sc_agg(hqf, src_all, cidx_t, attr_t):
    kern = pl.kernel(
        _sc_body,
        out_type=jax.ShapeDtypeStruct((_NQ, _RN, _QW), jnp.float32),
        mesh=plsc.VectorSubcoreMesh(core_axis_name="c", subcore_axis_name="s"),
        compiler_params=pltpu.CompilerParams(use_tc_tiling_on_sc=False),
        scratch_types=[
            pltpu.VMEM((_CB, _CH), jnp.int32),
            pltpu.VMEM((_CB, _CH), jnp.int32),
            pltpu.VMEM((_CB, _CH), jnp.float32),
            pltpu.VMEM((_NB, _CH, _QW), jnp.float32),
            pltpu.VMEM((_ZR, _QW), jnp.float32),
            pltpu.VMEM_SHARED((_RN, _QW), jnp.float32),
            pltpu.SemaphoreType.DMA,
            pltpu.SemaphoreType.DMA,
        ],
    )
    return kern(hqf, src_all, cidx_t, attr_t)


def kernel(x, edge_index, edge_type, edge_attr, W_in, b_in, g_in, be_in,
           Wg, bg, W1, b1, g1, bb1, W2, b2, Wt1, bt1, gt1, bbt1, Wt2, bt2):
    src = edge_index[0]
    dst = edge_index[1]
    cidx = dst * _R + edge_type
    attr = edge_attr[:, 0]

    pad = _EP - _E
    srcp = jnp.pad(src, (0, pad))
    cidxp = jnp.pad(cidx, (0, pad))
    attrp = jnp.pad(attr, (0, pad))

    qoff = (jnp.arange(_NQ, dtype=jnp.int32) * _N)[:, None]
    src_all = (srcp[None, :] + qoff).reshape(_NQ, _NS, _CPT, _CH)
    cidx_t = cidxp.reshape(_NS, _CPT, _CH)
    attr_t = attrp.reshape(_NS, _CPT, _CH)

    h = _proj(x, W_in, b_in, g_in, be_in)

    for l in range(2):
        hqf = h.reshape(_N, _NQ, _QW).transpose(1, 0, 2).reshape(_NQ * _N, _QW)
        a = _sc_agg(hqf, src_all, cidx_t, attr_t)
        a2 = a.reshape(_NQ, _N, _R * _QW)
        wq = jnp.stack([
            Wg[l, :, qq * _QW:(qq + 1) * _QW, :].reshape(_R * _QW, _H)
            for qq in range(_NQ)])
        h = _mid(a2, wq, bg[l])

    return _head(h, W1, b1, g1, bb1, W2, b2, Wt1, bt1, gt1, bbt1, Wt2, bt2)


# speedup vs baseline: 8.1934x; 1.0202x over previous
import functools

import jax
import jax.numpy as jnp
import numpy as np
from jax import lax
from jax.experimental import pallas as pl
from jax.experimental.pallas import tpu as pltpu
from jax.experimental.pallas import tpu_sc as plsc

_N = 10000
_E = 320000
_H = 128
_R = 5
_NQ = 4
_QW = 32
_NS = 16
_NC = 2
_CH = 128
_CPT = 160
_CB = 16
_EPT = _CH * _CPT
_EP = _NS * _EPT
_RN = _R * _N
_STRIPE = 3128
_LSTRIPE = _RN - 15 * _STRIPE
_ZR = 136
_NB = 4
_BM = 400


def _layer_norm(x, g, b):
    mu = jnp.mean(x, axis=-1, keepdims=True)
    var = jnp.mean(jnp.square(x - mu), axis=-1, keepdims=True)
    return (x - mu) / jnp.sqrt(var + 1e-5) * g + b


def _gelu(x):
    return x * 0.5 * (1.0 + lax.erf(x * np.float32(1.0 / np.sqrt(2.0))))


def _proj_body(x_ref, w_ref, b_ref, g_ref, be_ref, o_ref):
    h = jnp.dot(x_ref[...], w_ref[...], preferred_element_type=jnp.float32)
    o_ref[...] = _gelu(_layer_norm(h + b_ref[...], g_ref[...], be_ref[...]))


def _proj(x, w, b, g, be):
    return pl.pallas_call(
        _proj_body,
        grid=(_N // _BM,),
        in_specs=[
            pl.BlockSpec((_BM, _H), lambda i: (i, 0)),
            pl.BlockSpec((_H, _H), lambda i: (0, 0)),
            pl.BlockSpec((1, _H), lambda i: (0, 0)),
            pl.BlockSpec((1, _H), lambda i: (0, 0)),
            pl.BlockSpec((1, _H), lambda i: (0, 0)),
        ],
        out_specs=pl.BlockSpec((_BM, _H), lambda i: (i, 0)),
        out_shape=jax.ShapeDtypeStruct((_N, _H), jnp.float32),
    )(x, w, b.reshape(1, -1), g.reshape(1, -1), be.reshape(1, -1))


def _mid_body(a_ref, wq_ref, b_ref, o_ref):
    acc = jnp.zeros((_BM, _H), jnp.float32)
    for q in range(_NQ):
        acc += jnp.dot(a_ref[q], wq_ref[q], preferred_element_type=jnp.float32)
    o_ref[...] = _gelu(acc + b_ref[...])


def _mid(a2, wq, b):
    rw = _R * _QW
    return pl.pallas_call(
        _mid_body,
        grid=(_N // _BM,),
        in_specs=[
            pl.BlockSpec((_NQ, _BM, rw), lambda i: (0, i, 0)),
            pl.BlockSpec((_NQ, rw, _H), lambda i: (0, 0, 0)),
            pl.BlockSpec((1, _H), lambda i: (0, 0)),
        ],
        out_specs=pl.BlockSpec((_BM, _H), lambda i: (i, 0)),
        out_shape=jax.ShapeDtypeStruct((_N, _H), jnp.float32),
    )(a2, wq, b.reshape(1, -1))


def _head_body(h_ref, w1_ref, b1_ref, g1_ref, bb1_ref, w2_ref, b2_ref,
               wt1_ref, bt1_ref, gt1_ref, bbt1_ref, wt2_ref, bt2_ref,
               cat_ref, tax_ref):
    h = h_ref[...]
    z = _gelu(_layer_norm(
        jnp.dot(h, w1_ref[...], preferred_element_type=jnp.float32) + b1_ref[...],
        g1_ref[...], bb1_ref[...]))
    cat_ref[...] = jnp.dot(z, w2_ref[...], preferred_element_type=jnp.float32) + b2_ref[...]
    t = _gelu(_layer_norm(
        jnp.dot(h, wt1_ref[...], preferred_element_type=jnp.float32) + bt1_ref[...],
        gt1_ref[...], bbt1_ref[...]))
    tax_ref[...] = jnp.dot(t, wt2_ref[...], preferred_element_type=jnp.float32) + bt2_ref[...]


def _head(h, w1, b1, g1, bb1, w2, b2, wt1, bt1, gt1, bbt1, wt2, bt2):
    h2 = 2 * _H
    out = 400
    tax = 20
    return pl.pallas_call(
        _head_body,
        grid=(_N // _BM,),
        in_specs=[
            pl.BlockSpec((_BM, _H), lambda i: (i, 0)),
            pl.BlockSpec((_H, h2), lambda i: (0, 0)),
            pl.BlockSpec((1, h2), lambda i: (0, 0)),
            pl.BlockSpec((1, h2), lambda i: (0, 0)),
            pl.BlockSpec((1, h2), lambda i: (0, 0)),
            pl.BlockSpec((h2, out), lambda i: (0, 0)),
            pl.BlockSpec((1, out), lambda i: (0, 0)),
            pl.BlockSpec((_H, h2), lambda i: (0, 0)),
            pl.BlockSpec((1, h2), lambda i: (0, 0)),
            pl.BlockSpec((1, h2), lambda i: (0, 0)),
            pl.BlockSpec((1, h2), lambda i: (0, 0)),
            pl.BlockSpec((h2, tax), lambda i: (0, 0)),
            pl.BlockSpec((1, tax), lambda i: (0, 0)),
        ],
        out_specs=[
            pl.BlockSpec((_BM, out), lambda i: (i, 0)),
            pl.BlockSpec((_BM, tax), lambda i: (i, 0)),
        ],
        out_shape=[
            jax.ShapeDtypeStruct((_N, out), jnp.float32),
            jax.ShapeDtypeStruct((_N, tax), jnp.float32),
        ],
    )(h, w1, b1.reshape(1, -1), g1.reshape(1, -1), bb1.reshape(1, -1),
      w2, b2.reshape(1, -1), wt1, bt1.reshape(1, -1), gt1.reshape(1, -1),
      bbt1.reshape(1, -1), wt2, bt2.reshape(1, -1))


def _splat16(v, i):
    return lax.gather(
        v,
        jnp.full((16, 1), i, jnp.int32),
        lax.GatherDimensionNumbers(
            offset_dims=(), collapsed_slice_dims=(0,), start_index_map=(0,)),
        (1,),
        mode=lax.GatherScatterMode.PROMISE_IN_BOUNDS)


def _sc_body(hqf, src_all, cidx_t, attr_t, aq,
             src_v, cidx_v, attr_v, rows4, zbuf, acc, gsem, ssem):
    c = lax.axis_index("c")
    s = lax.axis_index("s")

    z16 = jnp.zeros((16,), jnp.float32)
    for i in range(_ZR):
        zbuf[i, 0:16] = z16
        zbuf[i, 16:32] = z16

    for qi in range(2):
        q = 2 * c + qi

        def _zero(k, carry):
            pltpu.sync_copy(zbuf, acc.at[pl.ds(s * _STRIPE + k * _ZR, _ZR)])
            return carry
        lax.fori_loop(0, jnp.where(s == _NS - 1, 22, 23), _zero, 0)

        @pl.when(s == _NS - 1)
        def _zero_tail():
            pltpu.sync_copy(zbuf.at[pl.ds(0, 88)],
                            acc.at[pl.ds(s * _STRIPE + 22 * _ZR, 88)])
        plsc.subcore_barrier()

        def _quad(qq, carry):
            is_stage = qq % (_CB // _NB) == 0

            @pl.when(jnp.logical_and(is_stage, qq > 0))
            def _flush():
                for b in range(_NB):
                    pltpu.make_async_copy(
                        rows4.at[b], acc.at[cidx_v.at[0]], ssem).wait()

            @pl.when(is_stage)
            def _stage():
                jg = qq // (_CB // _NB)
                pltpu.sync_copy(src_all.at[q, s, pl.ds(jg * _CB, _CB)], src_v)
                pltpu.sync_copy(cidx_t.at[s, pl.ds(jg * _CB, _CB)], cidx_v)
                pltpu.sync_copy(attr_t.at[s, pl.ds(jg * _CB, _CB)], attr_v)

            jr = (qq % (_CB // _NB)) * _NB
            for b in range(_NB):
                @pl.when(jnp.logical_not(is_stage))
                def _drain():
                    pltpu.make_async_copy(
                        rows4.at[b], acc.at[cidx_v.at[0]], ssem).wait()
                pltpu.async_copy(hqf.at[src_v.at[jr + b]], rows4.at[b], gsem)
            for b in range(_NB):
                pltpu.make_async_copy(
                    hqf.at[src_v.at[jr + b]], rows4.at[b], gsem).wait()
                for g in range(_CH // 16):
                    a16 = attr_v[jr + b, pl.ds(g * 16, 16)]
                    for i in range(16):
                        e = g * 16 + i
                        spl = _splat16(a16, i)
                        rows4[b, e, 0:16] = rows4[b, e, 0:16] * spl
                        rows4[b, e, 16:32] = rows4[b, e, 16:32] * spl
                pltpu.async_copy(
                    rows4.at[b], acc.at[cidx_v.at[jr + b]], ssem, add=True)
            return carry
        lax.fori_loop(0, _CPT // _NB, _quad, 0)
        for b in range(_NB):
            pltpu.make_async_copy(
                rows4.at[b], acc.at[cidx_v.at[0]], ssem).wait()
        plsc.subcore_barrier()

        @pl.when(s < _NS - 1)
        def _writeout():
            pltpu.sync_copy(acc.at[pl.ds(s * _STRIPE, _STRIPE)],
                            aq.at[q, pl.ds(s * _STRIPE, _STRIPE)])

        @pl.when(s == _NS - 1)
        def _writeout_tail():
            pltpu.sync_copy(acc.at[pl.ds(s * _STRIPE, _LSTRIPE)],
                            aq.at[q, pl.ds(s * _STRIPE, _LSTRIPE)])
        plsc.subcore_barrier()


def _sc_agg(hqf, src_all, cidx_t, attr_t):
    kern = pl.kernel(
        _sc_body,
        out_type=jax.ShapeDtypeStruct((_NQ, _RN, _QW), jnp.float32),
        mesh=plsc.VectorSubcoreMesh(core_axis_name="c", subcore_axis_name="s"),
        compiler_params=pltpu.CompilerParams(use_tc_tiling_on_sc=False),
        scratch_types=[
            pltpu.VMEM((_CB, _CH), jnp.int32),
            pltpu.VMEM((_CB, _CH), jnp.int32),
            pltpu.VMEM((_CB, _CH), jnp.float32),
            pltpu.VMEM((_NB, _CH, _QW), jnp.float32),
            pltpu.VMEM((_ZR, _QW), jnp.float32),
            pltpu.VMEM_SHARED((_RN, _QW), jnp.float32),
            pltpu.SemaphoreType.DMA,
            pltpu.SemaphoreType.DMA,
        ],
    )
    return kern(hqf, src_all, cidx_t, attr_t)


def kernel(x, edge_index, edge_type, edge_attr, W_in, b_in, g_in, be_in,
           Wg, bg, W1, b1, g1, bb1, W2, b2, Wt1, bt1, gt1, bbt1, Wt2, bt2):
    src = edge_index[0]
    dst = edge_index[1]
    cidx = dst * _R + edge_type
    attr = edge_attr[:, 0]

    pad = _EP - _E
    srcp = jnp.pad(src, (0, pad))
    cidxp = jnp.pad(cidx, (0, pad))
    attrp = jnp.pad(attr, (0, pad))

    qoff = (jnp.arange(_NQ, dtype=jnp.int32) * _N)[:, None]
    src_all = (srcp[None, :] + qoff).reshape(_NQ, _NS, _CPT, _CH)
    cidx_t = cidxp.reshape(_NS, _CPT, _CH)
    attr_t = attrp.reshape(_NS, _CPT, _CH)

    h = _proj(x, W_in, b_in, g_in, be_in)

    for l in range(2):
        hqf = h.reshape(_N, _NQ, _QW).transpose(1, 0, 2).reshape(_NQ * _N, _QW)
        a = _sc_agg(hqf, src_all, cidx_t, attr_t)
        a2 = a.reshape(_NQ, _N, _R * _QW)
        wq = jnp.stack([
            Wg[l, :, qq * _QW:(qq + 1) * _QW, :].reshape(_R * _QW, _H)
            for qq in range(_NQ)])
        h = _mid(a2, wq, bg[l])

    return _head(h, W1, b1, g1, bb1, W2, b2, Wt1, bt1, gt1, bbt1, Wt2, bt2)


# speedup vs baseline: 9.9000x; 1.2083x over previous
import functools

import jax
import jax.numpy as jnp
import numpy as np
from jax import lax
from jax.experimental import pallas as pl
from jax.experimental.pallas import tpu as pltpu
from jax.experimental.pallas import tpu_sc as plsc

_N = 10000
_E = 320000
_H = 128
_R = 5
_NQ = 4
_QW = 32
_NS = 16
_NC = 2
_CH = 128
_CPT = 160
_CB = 4
_EPT = _CH * _CPT
_EP = _NS * _EPT
_RN = _R * _N
_STRIPE = 3128
_LSTRIPE = _RN - 15 * _STRIPE
_NB = 2
_BM = 400


def _layer_norm(x, g, b):
    mu = jnp.mean(x, axis=-1, keepdims=True)
    var = jnp.mean(jnp.square(x - mu), axis=-1, keepdims=True)
    return (x - mu) / jnp.sqrt(var + 1e-5) * g + b


def _gelu(x):
    return x * 0.5 * (1.0 + lax.erf(x * np.float32(1.0 / np.sqrt(2.0))))


def _proj_body(x_ref, w_ref, b_ref, g_ref, be_ref, o_ref):
    h = jnp.dot(x_ref[...], w_ref[...], preferred_element_type=jnp.float32)
    o_ref[...] = _gelu(_layer_norm(h + b_ref[...], g_ref[...], be_ref[...]))


def _proj(x, w, b, g, be):
    return pl.pallas_call(
        _proj_body,
        grid=(_N // _BM,),
        in_specs=[
            pl.BlockSpec((_BM, _H), lambda i: (i, 0)),
            pl.BlockSpec((_H, _H), lambda i: (0, 0)),
            pl.BlockSpec((1, _H), lambda i: (0, 0)),
            pl.BlockSpec((1, _H), lambda i: (0, 0)),
            pl.BlockSpec((1, _H), lambda i: (0, 0)),
        ],
        out_specs=pl.BlockSpec((_BM, _H), lambda i: (i, 0)),
        out_shape=jax.ShapeDtypeStruct((_N, _H), jnp.float32),
    )(x, w, b.reshape(1, -1), g.reshape(1, -1), be.reshape(1, -1))


def _mid_body(a_ref, wq_ref, b_ref, o_ref):
    acc = jnp.zeros((_BM, _H), jnp.float32)
    for q in range(_NQ):
        acc += jnp.dot(a_ref[q], wq_ref[q], preferred_element_type=jnp.float32)
    o_ref[...] = _gelu(acc + b_ref[...])


def _mid(a2, wq, b):
    rw = _R * _QW
    return pl.pallas_call(
        _mid_body,
        grid=(_N // _BM,),
        in_specs=[
            pl.BlockSpec((_NQ, _BM, rw), lambda i: (0, i, 0)),
            pl.BlockSpec((_NQ, rw, _H), lambda i: (0, 0, 0)),
            pl.BlockSpec((1, _H), lambda i: (0, 0)),
        ],
        out_specs=pl.BlockSpec((_BM, _H), lambda i: (i, 0)),
        out_shape=jax.ShapeDtypeStruct((_N, _H), jnp.float32),
    )(a2, wq, b.reshape(1, -1))


def _head_body(h_ref, w1_ref, b1_ref, g1_ref, bb1_ref, w2_ref, b2_ref,
               wt1_ref, bt1_ref, gt1_ref, bbt1_ref, wt2_ref, bt2_ref,
               cat_ref, tax_ref):
    h = h_ref[...]
    z = _gelu(_layer_norm(
        jnp.dot(h, w1_ref[...], preferred_element_type=jnp.float32) + b1_ref[...],
        g1_ref[...], bb1_ref[...]))
    cat_ref[...] = jnp.dot(z, w2_ref[...], preferred_element_type=jnp.float32) + b2_ref[...]
    t = _gelu(_layer_norm(
        jnp.dot(h, wt1_ref[...], preferred_element_type=jnp.float32) + bt1_ref[...],
        gt1_ref[...], bbt1_ref[...]))
    tax_ref[...] = jnp.dot(t, wt2_ref[...], preferred_element_type=jnp.float32) + bt2_ref[...]


def _head(h, w1, b1, g1, bb1, w2, b2, wt1, bt1, gt1, bbt1, wt2, bt2):
    h2 = 2 * _H
    out = 400
    tax = 20
    return pl.pallas_call(
        _head_body,
        grid=(_N // _BM,),
        in_specs=[
            pl.BlockSpec((_BM, _H), lambda i: (i, 0)),
            pl.BlockSpec((_H, h2), lambda i: (0, 0)),
            pl.BlockSpec((1, h2), lambda i: (0, 0)),
            pl.BlockSpec((1, h2), lambda i: (0, 0)),
            pl.BlockSpec((1, h2), lambda i: (0, 0)),
            pl.BlockSpec((h2, out), lambda i: (0, 0)),
            pl.BlockSpec((1, out), lambda i: (0, 0)),
            pl.BlockSpec((_H, h2), lambda i: (0, 0)),
            pl.BlockSpec((1, h2), lambda i: (0, 0)),
            pl.BlockSpec((1, h2), lambda i: (0, 0)),
            pl.BlockSpec((1, h2), lambda i: (0, 0)),
            pl.BlockSpec((h2, tax), lambda i: (0, 0)),
            pl.BlockSpec((1, tax), lambda i: (0, 0)),
        ],
        out_specs=[
            pl.BlockSpec((_BM, out), lambda i: (i, 0)),
            pl.BlockSpec((_BM, tax), lambda i: (i, 0)),
        ],
        out_shape=[
            jax.ShapeDtypeStruct((_N, out), jnp.float32),
            jax.ShapeDtypeStruct((_N, tax), jnp.float32),
        ],
    )(h, w1, b1.reshape(1, -1), g1.reshape(1, -1), bb1.reshape(1, -1),
      w2, b2.reshape(1, -1), wt1, bt1.reshape(1, -1), gt1.reshape(1, -1),
      bbt1.reshape(1, -1), wt2, bt2.reshape(1, -1))


def _splat16(v, i):
    return lax.gather(
        v,
        jnp.full((16, 1), i, jnp.int32),
        lax.GatherDimensionNumbers(
            offset_dims=(), collapsed_slice_dims=(0,), start_index_map=(0,)),
        (1,),
        mode=lax.GatherScatterMode.PROMISE_IN_BOUNDS)


def _sc_body(hq, src_t, cidx_t, attr_t, aq,
             src_v, cidx_v, attr_v, rows4, acc, htab, gsem, ssem):
    c = lax.axis_index("c")
    s = lax.axis_index("s")

    for qi in range(2):
        q = 2 * c + qi

        z16 = jnp.zeros((16,), jnp.float32)
        for i in range(_CH):
            rows4[0, i, 0:16] = z16
            rows4[0, i, 16:32] = z16

        @pl.when(s == 0)
        def _load_table():
            pltpu.sync_copy(hq.at[q], htab)

        def _zero(k, carry):
            pltpu.sync_copy(rows4.at[0],
                            acc.at[pl.ds(s * _STRIPE + k * _CH, _CH)])
            return carry
        lax.fori_loop(0, 24, _zero, 0)

        @pl.when(s < _NS - 1)
        def _zero_tail():
            pltpu.sync_copy(rows4.at[0, pl.ds(0, 56)],
                            acc.at[pl.ds(s * _STRIPE + 24 * _CH, 56)])

        @pl.when(s == _NS - 1)
        def _zero_tail2():
            pltpu.sync_copy(rows4.at[0, pl.ds(0, 8)],
                            acc.at[pl.ds(s * _STRIPE + 24 * _CH, 8)])
        plsc.subcore_barrier()

        def _quad(qq, carry):
            is_stage = qq % (_CB // _NB) == 0

            @pl.when(jnp.logical_and(is_stage, qq > 0))
            def _flush():
                for b in range(_NB):
                    pltpu.make_async_copy(
                        rows4.at[b], acc.at[cidx_v.at[0]], ssem).wait()

            @pl.when(is_stage)
            def _stage():
                jg = qq // (_CB // _NB)
                pltpu.sync_copy(src_t.at[s, pl.ds(jg * _CB, _CB)], src_v)
                pltpu.sync_copy(cidx_t.at[s, pl.ds(jg * _CB, _CB)], cidx_v)
                pltpu.sync_copy(attr_t.at[s, pl.ds(jg * _CB, _CB)], attr_v)

            jr = (qq % (_CB // _NB)) * _NB
            for b in range(_NB):
                @pl.when(jnp.logical_not(is_stage))
                def _drain():
                    pltpu.make_async_copy(
                        rows4.at[b], acc.at[cidx_v.at[0]], ssem).wait()
                pltpu.async_copy(htab.at[src_v.at[jr + b]], rows4.at[b], gsem)
            for b in range(_NB):
                pltpu.make_async_copy(
                    htab.at[src_v.at[jr + b]], rows4.at[b], gsem).wait()
                for g in range(_CH // 16):
                    a16 = attr_v[jr + b, pl.ds(g * 16, 16)]
                    for i in range(16):
                        e = g * 16 + i
                        spl = _splat16(a16, i)
                        rows4[b, e, 0:16] = rows4[b, e, 0:16] * spl
                        rows4[b, e, 16:32] = rows4[b, e, 16:32] * spl
                pltpu.async_copy(
                    rows4.at[b], acc.at[cidx_v.at[jr + b]], ssem, add=True)
            return carry
        lax.fori_loop(0, _CPT // _NB, _quad, 0)
        for b in range(_NB):
            pltpu.make_async_copy(
                rows4.at[b], acc.at[cidx_v.at[0]], ssem).wait()
        plsc.subcore_barrier()

        @pl.when(s < _NS - 1)
        def _writeout():
            pltpu.sync_copy(acc.at[pl.ds(s * _STRIPE, _STRIPE)],
                            aq.at[q, pl.ds(s * _STRIPE, _STRIPE)])

        @pl.when(s == _NS - 1)
        def _writeout_tail():
            pltpu.sync_copy(acc.at[pl.ds(s * _STRIPE, _LSTRIPE)],
                            aq.at[q, pl.ds(s * _STRIPE, _LSTRIPE)])
        plsc.subcore_barrier()


def _sc_agg(hq, src_t, cidx_t, attr_t):
    kern = pl.kernel(
        _sc_body,
        out_type=jax.ShapeDtypeStruct((_NQ, _RN, _QW), jnp.float32),
        mesh=plsc.VectorSubcoreMesh(core_axis_name="c", subcore_axis_name="s"),
        compiler_params=pltpu.CompilerParams(use_tc_tiling_on_sc=False),
        scratch_types=[
            pltpu.VMEM((_CB, _CH), jnp.int32),
            pltpu.VMEM((_CB, _CH), jnp.int32),
            pltpu.VMEM((_CB, _CH), jnp.float32),
            pltpu.VMEM((_NB, _CH, _QW), jnp.float32),
            pltpu.VMEM_SHARED((_RN, _QW), jnp.float32),
            pltpu.VMEM_SHARED((_N, _QW), jnp.float32),
            pltpu.SemaphoreType.DMA,
            pltpu.SemaphoreType.DMA,
        ],
    )
    return kern(hq, src_t, cidx_t, attr_t)


def kernel(x, edge_index, edge_type, edge_attr, W_in, b_in, g_in, be_in,
           Wg, bg, W1, b1, g1, bb1, W2, b2, Wt1, bt1, gt1, bbt1, Wt2, bt2):
    src = edge_index[0]
    dst = edge_index[1]
    cidx = dst * _R + edge_type
    attr = edge_attr[:, 0]

    pad = _EP - _E
    srcp = jnp.pad(src, (0, pad))
    cidxp = jnp.pad(cidx, (0, pad))
    attrp = jnp.pad(attr, (0, pad))

    src_t = srcp.reshape(_NS, _CPT, _CH)
    cidx_t = cidxp.reshape(_NS, _CPT, _CH)
    attr_t = attrp.reshape(_NS, _CPT, _CH)

    h = _proj(x, W_in, b_in, g_in, be_in)

    for l in range(2):
        hq = h.reshape(_N, _NQ, _QW).transpose(1, 0, 2)
        a = _sc_agg(hq, src_t, cidx_t, attr_t)
        a2 = a.reshape(_NQ, _N, _R * _QW)
        wq = jnp.stack([
            Wg[l, :, qq * _QW:(qq + 1) * _QW, :].reshape(_R * _QW, _H)
            for qq in range(_NQ)])
        h = _mid(a2, wq, bg[l])

    return _head(h, W1, b1, g1, bb1, W2, b2, Wt1, bt1, gt1, bbt1, Wt2, bt2)
